# baseline (speedup 1.0000x reference)
import functools

import jax
import jax.numpy as jnp
from jax import lax
from jax.experimental import pallas as pl
from jax.experimental.pallas import tpu as pltpu
from jax.experimental.pallas import tpu_sc as plsc

N_NODES = 10000
IN_DIM = 128
HID_DIM = 256
NUM_ASPECT = 8
ASPECT_DIM = 32
BATCH = 4096

DEG_W = 16
N_PAD = 10112
NCOLSUM = 16
COLSUM_BASE = N_PAD - NCOLSUM
DUMMY = N_NODES
NDUMMY = 64

NC, NS = 2, 16
NW = NC * NS
CHUNK = 96
ROWS_PER_TILE = N_PAD // NS

NBUF = 2
IDX_BITS = 14

PAIR_PAD = 9216
PAIR_CHUNKS = PAIR_PAD // (NW * CHUNK)
PAIR_PER_TILE = PAIR_PAD // NW
PAIR_CHUNKS2 = PAIR_PAD // (NS * CHUNK)
PAIR_PER_TILE2 = PAIR_PAD // NS


def _edge_chunks(n_edges_total):
    per_w = -(-n_edges_total // (NW * CHUNK))
    return -(-per_w // NBUF) * NBUF


def _make_edge_kernel(n_chunks):
    mesh = plsc.VectorSubcoreMesh(core_axis_name="c", subcore_axis_name="s")

    @functools.partial(
        pl.kernel,
        mesh=mesh,
        compiler_params=pltpu.CompilerParams(use_tc_tiling_on_sc=False),
        out_type=(
            jax.ShapeDtypeStruct((NC, PAIR_PAD, IN_DIM), jnp.float32),
            jax.ShapeDtypeStruct((NC, PAIR_PAD, DEG_W), jnp.float32),
            jax.ShapeDtypeStruct((PAIR_PAD, IN_DIM), jnp.float32),
            jax.ShapeDtypeStruct((NC, NCOLSUM, IN_DIM), jnp.float32),
            jax.ShapeDtypeStruct((NC, N_PAD, IN_DIM), jnp.float32),
            jax.ShapeDtypeStruct((NC, N_PAD, DEG_W), jnp.float32),
        ),
        scratch_types=[
            pltpu.VMEM((n_chunks, CHUNK), jnp.int32),
            pltpu.VMEM((NBUF, CHUNK), jnp.int32),
            pltpu.VMEM((1, CHUNK), jnp.int32),
            pltpu.VMEM((PAIR_CHUNKS2, CHUNK), jnp.int32),
            pltpu.VMEM((NBUF, CHUNK, IN_DIM), jnp.float32),
            pltpu.VMEM((CHUNK, DEG_W), jnp.float32),
            pltpu.VMEM_SHARED((N_PAD, IN_DIM), jnp.float32),
            pltpu.VMEM_SHARED((N_PAD, DEG_W), jnp.float32),
            pltpu.SemaphoreType.DMA,
        ],
    )
    def edge_kernel(feat_hbm, edges_hbm, zerof_hbm, zerod_hbm, ones_hbm,
                    pidx2_hbm,
                    pairf_hbm, paird_hbm, pairn_hbm, tails_hbm,
                    outf_hbm, outd_hbm,
                    packed_v, usrc_v, udst_v, pidx2_v, rows_v, ones_v,
                    accf_sh, accd_sh, sem):
        cid = lax.axis_index("c")
        sid = lax.axis_index("s")
        wid = sid * NC + cid
        r0 = sid * ROWS_PER_TILE
        pltpu.sync_copy(zerof_hbm, accf_sh.at[pl.ds(r0, ROWS_PER_TILE)])
        pltpu.sync_copy(zerod_hbm, accd_sh.at[pl.ds(r0, ROWS_PER_TILE)])
        pltpu.sync_copy(edges_hbm.at[wid], packed_v)
        pltpu.sync_copy(ones_hbm, ones_v)
        plsc.subcore_barrier()

        def unpack_src(j, b):
            for k in range(CHUNK // 16):
                w = packed_v[j, pl.ds(k * 16, 16)]
                usrc_v[b, pl.ds(k * 16, 16)] = w & ((1 << IDX_BITS) - 1)

        def unpack_dst(j):
            for k in range(CHUNK // 16):
                w = packed_v[j, pl.ds(k * 16, 16)]
                udst_v[0, pl.ds(k * 16, 16)] = w >> IDX_BITS

        for b in range(NBUF):
            unpack_src(b, b)
            pltpu.async_copy(feat_hbm.at[usrc_v.at[b]], rows_v.at[b], sem)

        def group(g, carry):
            for b in range(NBUF):
                j = g * NBUF + b
                pltpu.make_async_copy(
                    feat_hbm.at[usrc_v.at[b]], rows_v.at[b], sem).wait()
                unpack_dst(j)
                pltpu.sync_copy(rows_v.at[b], accf_sh.at[udst_v.at[0]],
                                add=True)
                pltpu.sync_copy(ones_v, accd_sh.at[udst_v.at[0]], add=True)
                nj = j + NBUF

                @pl.when(nj < n_chunks)
                def _issue():
                    unpack_src(nj, b)
                    pltpu.async_copy(
                        feat_hbm.at[usrc_v.at[b]], rows_v.at[b], sem)
            return carry

        lax.fori_loop(0, n_chunks // NBUF, group, 0)
        plsc.subcore_barrier()

        pltpu.sync_copy(accf_sh.at[pl.ds(r0, ROWS_PER_TILE)],
                        outf_hbm.at[cid, pl.ds(r0, ROWS_PER_TILE)])
        pltpu.sync_copy(accd_sh.at[pl.ds(r0, ROWS_PER_TILE)],
                        outd_hbm.at[cid, pl.ds(r0, ROWS_PER_TILE)])

        @pl.when(sid == NS - 1)
        def _tails():
            pltpu.sync_copy(accf_sh.at[pl.ds(COLSUM_BASE, NCOLSUM)],
                            tails_hbm.at[cid])

        pltpu.sync_copy(pidx2_hbm.at[sid], pidx2_v)
        plsc.subcore_barrier()
        for c in range(PAIR_CHUNKS2):
            base = sid * PAIR_PER_TILE2 + c * CHUNK
            pltpu.async_copy(outf_hbm.at[cid].at[pidx2_v.at[c]], rows_v.at[0],
                             sem).wait()
            pltpu.sync_copy(rows_v.at[0], pairf_hbm.at[cid, pl.ds(base, CHUNK)])
            pltpu.async_copy(outd_hbm.at[cid].at[pidx2_v.at[c]], ones_v,
                             sem).wait()
            pltpu.sync_copy(ones_v, paird_hbm.at[cid, pl.ds(base, CHUNK)])
        for c in range(PAIR_CHUNKS):
            base = wid * PAIR_PER_TILE + c * CHUNK
            pltpu.async_copy(
                feat_hbm.at[pidx2_v.at[cid * PAIR_CHUNKS + c]], rows_v.at[1],
                sem).wait()
            pltpu.sync_copy(rows_v.at[1], pairn_hbm.at[pl.ds(base, CHUNK)])

    return edge_kernel


def _head_body(p1f0, p1f1, p1d0, p1d1, p1n, p2f0, p2f1, p2d0, p2d1, p2n,
               t0_ref, t1_ref, w1t_ref, b1_ref, wasp_ref, g_ref, gt_ref,
               wat_ref, wmg_ref, wc1_ref, wc2_ref, bc_ref, out_ref):
    tl = t0_ref[0] + t1_ref[0]
    mu = jnp.sum(tl, axis=0, keepdims=True) * (1.0 / N_NODES)

    def xrows(f0, f1, d0, d1, nrows):
        s = f0[0] + f1[0]
        deg = d0[0, :, 0:1] + d1[0, :, 0:1]
        t = (nrows[...] + mu) * (s + deg * mu) / jnp.maximum(deg, 1.0)
        return jnp.dot(t, w1t_ref[...],
                       preferred_element_type=jnp.float32) + b1_ref[0:1, :]

    def merge(p):
        q = jnp.dot(p, wasp_ref[...], preferred_element_type=jnp.float32)
        l = jnp.dot(q * wat_ref[0:1, :], g_ref[...],
                    preferred_element_type=jnp.float32)
        m = l - jnp.max(l, axis=1, keepdims=True)
        e = jnp.exp(m)
        sm = e / jnp.sum(e, axis=1, keepdims=True)
        ab = jnp.dot(sm, gt_ref[...], preferred_element_type=jnp.float32)
        return jnp.dot(ab * q * wmg_ref[0:1, :], g_ref[...],
                       preferred_element_type=jnp.float32)

    m1 = merge(xrows(p1f0, p1f1, p1d0, p1d1, p1n))
    m2 = merge(xrows(p2f0, p2f1, p2d0, p2d1, p2n))
    o = jnp.dot(m1, wc1_ref[...], preferred_element_type=jnp.float32) \
        + jnp.dot(m2, wc2_ref[...], preferred_element_type=jnp.float32) \
        + bc_ref[0:1, :]
    mx = jnp.max(o, axis=1, keepdims=True)
    out_ref[...] = o - mx - jnp.log(jnp.sum(jnp.exp(o - mx), axis=1,
                                            keepdims=True))


def kernel(input, node_feature, edge_index, W1, b1, Wasp, Wattn, battn,
           Wmerge, bmerge, Wc, bc):
    f32 = jnp.float32
    n_edges = edge_index.shape[1]

    fake_src = jnp.arange(N_NODES, dtype=jnp.int32)
    fake_dst = COLSUM_BASE + (fake_src % NCOLSUM)
    total = n_edges + N_NODES
    n_chunks = _edge_chunks(total)
    e_pad = NW * n_chunks * CHUNK
    npad = e_pad - total
    pad_src = jnp.zeros((npad,), jnp.int32)
    pad_dst = DUMMY + (jnp.arange(npad, dtype=jnp.int32) % NDUMMY)
    src_all = jnp.concatenate([edge_index[0], fake_src, pad_src])
    dst_all = jnp.concatenate([edge_index[1], fake_dst, pad_dst])
    packed = ((dst_all << IDX_BITS) | src_all) \
        .reshape(n_chunks, NW, CHUNK).transpose(1, 0, 2)

    idx_flat = jnp.concatenate(
        [input[:, 0], input[:, 1],
         jnp.zeros((PAIR_PAD - 2 * BATCH,), input.dtype)]).astype(jnp.int32)
    pidx2 = idx_flat.reshape(NS, PAIR_CHUNKS2, CHUNK)

    zero_f = jnp.zeros((ROWS_PER_TILE, IN_DIM), f32)
    zero_d = jnp.zeros((ROWS_PER_TILE, DEG_W), f32)
    ones_rows = jnp.zeros((CHUNK, DEG_W), f32).at[:, 0].set(1.0)
    pairf, paird, pairn, tails, _outf, _outd = _make_edge_kernel(n_chunks)(
        node_feature, packed, zero_f, zero_d, ones_rows, pidx2)

    hid = NUM_ASPECT * ASPECT_DIM
    g = (jnp.arange(hid)[:, None] // ASPECT_DIM
         == jnp.arange(NUM_ASPECT)[None, :]).astype(f32)
    gt = g.T
    wat = jnp.broadcast_to(jnp.tile(Wattn[0], NUM_ASPECT)[None, :], (8, hid))
    wmg = jnp.broadcast_to(jnp.tile(Wmerge[0], NUM_ASPECT)[None, :], (8, hid))
    wc1 = Wc[:, :NUM_ASPECT].T
    wc2 = Wc[:, NUM_ASPECT:].T
    bc_eff = bc + bmerge[0] * jnp.sum(Wc, axis=1)
    bcb = jnp.broadcast_to(bc_eff[None, :], (8, 2))
    w1t = W1.T
    b1b = jnp.broadcast_to(b1[None, :], (8, HID_DIM))

    bblk = 512
    bgrid = BATCH // bblk
    off2 = BATCH // bblk
    out = pl.pallas_call(
        _head_body,
        grid=(bgrid,),
        in_specs=[
            pl.BlockSpec((1, bblk, IN_DIM), lambda i: (0, i, 0)),
            pl.BlockSpec((1, bblk, IN_DIM), lambda i: (1, i, 0)),
            pl.BlockSpec((1, bblk, DEG_W), lambda i: (0, i, 0)),
            pl.BlockSpec((1, bblk, DEG_W), lambda i: (1, i, 0)),
            pl.BlockSpec((bblk, IN_DIM), lambda i: (i, 0)),
            pl.BlockSpec((1, bblk, IN_DIM), lambda i: (0, off2 + i, 0)),
            pl.BlockSpec((1, bblk, IN_DIM), lambda i: (1, off2 + i, 0)),
            pl.BlockSpec((1, bblk, DEG_W), lambda i: (0, off2 + i, 0)),
            pl.BlockSpec((1, bblk, DEG_W), lambda i: (1, off2 + i, 0)),
            pl.BlockSpec((bblk, IN_DIM), lambda i: (off2 + i, 0)),
            pl.BlockSpec((1, NCOLSUM, IN_DIM), lambda i: (0, 0, 0)),
            pl.BlockSpec((1, NCOLSUM, IN_DIM), lambda i: (1, 0, 0)),
            pl.BlockSpec((IN_DIM, HID_DIM), lambda i: (0, 0)),
            pl.BlockSpec((8, HID_DIM), lambda i: (0, 0)),
            pl.BlockSpec((HID_DIM, hid), lambda i: (0, 0)),
            pl.BlockSpec((hid, NUM_ASPECT), lambda i: (0, 0)),
            pl.BlockSpec((NUM_ASPECT, hid), lambda i: (0, 0)),
            pl.BlockSpec((8, hid), lambda i: (0, 0)),
            pl.BlockSpec((8, hid), lambda i: (0, 0)),
            pl.BlockSpec((NUM_ASPECT, 2), lambda i: (0, 0)),
            pl.BlockSpec((NUM_ASPECT, 2), lambda i: (0, 0)),
            pl.BlockSpec((8, 2), lambda i: (0, 0)),
        ],
        out_specs=pl.BlockSpec((bblk, 2), lambda i: (i, 0)),
        out_shape=jax.ShapeDtypeStruct((BATCH, 2), f32),
    )(pairf, pairf, paird, paird, pairn, pairf, pairf, paird, paird, pairn,
      tails, tails, w1t, b1b, Wasp.T, g, gt, wat, wmg, wc1, wc2, bcb)
    return out


# speedup vs baseline: 1.3365x; 1.3365x over previous
import functools

import jax
import jax.numpy as jnp
from jax import lax
from jax.experimental import pallas as pl
from jax.experimental.pallas import tpu as pltpu
from jax.experimental.pallas import tpu_sc as plsc

N_NODES = 10000
IN_DIM = 128
HID_DIM = 256
NUM_ASPECT = 8
ASPECT_DIM = 32
BATCH = 4096

DEG_W = 16
N_PAD = 10112
NCOLSUM = 16
COLSUM_BASE = N_PAD - NCOLSUM
DUMMY = N_NODES
NDUMMY = 64

NC, NS = 2, 16
NW = NC * NS
CHUNK = 96
PCHUNK = 128
ROWS_PER_TILE = N_PAD // NS


NBUF = 2
IDX_BITS = 14


def _edge_chunks(n_edges_total):
    per_w = -(-n_edges_total // (NW * CHUNK))
    return -(-per_w // NBUF) * NBUF


def _make_edge_scatter(n_chunks):
    mesh = plsc.VectorSubcoreMesh(core_axis_name="c", subcore_axis_name="s")

    @functools.partial(
        pl.kernel,
        mesh=mesh,
        compiler_params=pltpu.CompilerParams(use_tc_tiling_on_sc=False),
        out_type=(jax.ShapeDtypeStruct((NC, N_PAD, IN_DIM), jnp.float32),
                  jax.ShapeDtypeStruct((NC, N_PAD, DEG_W), jnp.float32)),
        scratch_types=[
            pltpu.VMEM((n_chunks, CHUNK), jnp.int32),
            pltpu.VMEM((NBUF, CHUNK), jnp.int32),
            pltpu.VMEM((1, CHUNK), jnp.int32),
            pltpu.VMEM((NBUF, CHUNK, IN_DIM), jnp.float32),
            pltpu.VMEM((CHUNK, DEG_W), jnp.float32),
            pltpu.VMEM_SHARED((N_PAD, IN_DIM), jnp.float32),
            pltpu.VMEM_SHARED((N_PAD, DEG_W), jnp.float32),
            pltpu.SemaphoreType.DMA,
        ],
    )
    def edge_scatter(feat_hbm, edges_hbm, zerof_hbm, zerod_hbm, ones_hbm,
                     outf_hbm, outd_hbm,
                     packed_v, usrc_v, udst_v, rows_v, ones_v,
                     accf_sh, accd_sh, gsem):
        cid = lax.axis_index("c")
        sid = lax.axis_index("s")
        wid = sid * NC + cid
        r0 = sid * ROWS_PER_TILE
        pltpu.sync_copy(zerof_hbm, accf_sh.at[pl.ds(r0, ROWS_PER_TILE)])
        pltpu.sync_copy(zerod_hbm, accd_sh.at[pl.ds(r0, ROWS_PER_TILE)])
        pltpu.sync_copy(edges_hbm.at[wid], packed_v)
        pltpu.sync_copy(ones_hbm, ones_v)
        plsc.subcore_barrier()

        def unpack_src(j, b):
            for k in range(CHUNK // 16):
                w = packed_v[j, pl.ds(k * 16, 16)]
                usrc_v[b, pl.ds(k * 16, 16)] = w & ((1 << IDX_BITS) - 1)

        def unpack_dst(j):
            for k in range(CHUNK // 16):
                w = packed_v[j, pl.ds(k * 16, 16)]
                udst_v[0, pl.ds(k * 16, 16)] = w >> IDX_BITS

        for b in range(NBUF):
            unpack_src(b, b)
            pltpu.async_copy(feat_hbm.at[usrc_v.at[b]], rows_v.at[b], gsem)

        def group(g, carry):
            for b in range(NBUF):
                j = g * NBUF + b
                pltpu.make_async_copy(
                    feat_hbm.at[usrc_v.at[b]], rows_v.at[b], gsem).wait()
                unpack_dst(j)
                pltpu.sync_copy(rows_v.at[b], accf_sh.at[udst_v.at[0]],
                                add=True)
                pltpu.sync_copy(ones_v, accd_sh.at[udst_v.at[0]], add=True)
                nj = j + NBUF

                @pl.when(nj < n_chunks)
                def _issue():
                    unpack_src(nj, b)
                    pltpu.async_copy(
                        feat_hbm.at[usrc_v.at[b]], rows_v.at[b], gsem)
            return carry

        lax.fori_loop(0, n_chunks // NBUF, group, 0)
        plsc.subcore_barrier()
        pltpu.sync_copy(accf_sh.at[pl.ds(r0, ROWS_PER_TILE)],
                        outf_hbm.at[cid, pl.ds(r0, ROWS_PER_TILE)])
        pltpu.sync_copy(accd_sh.at[pl.ds(r0, ROWS_PER_TILE)],
                        outd_hbm.at[cid, pl.ds(r0, ROWS_PER_TILE)])

    return edge_scatter


def _make_pair_gather():
    mesh = plsc.VectorSubcoreMesh(core_axis_name="c", subcore_axis_name="s")
    n_chunks = (2 * BATCH) // (NW * PCHUNK)

    @functools.partial(
        pl.kernel,
        mesh=mesh,
        out_type=jax.ShapeDtypeStruct((2 * BATCH, HID_DIM), jnp.float32),
        scratch_types=[
            pltpu.VMEM((n_chunks, PCHUNK), jnp.int32),
            pltpu.VMEM((PCHUNK, HID_DIM), jnp.float32),
            pltpu.SemaphoreType.DMA,
        ],
    )
    def pair_gather(x_hbm, idx_hbm, out_hbm, idx_v, rows_v, sem):
        cid = lax.axis_index("c")
        sid = lax.axis_index("s")
        wid = sid * NC + cid
        pltpu.sync_copy(idx_hbm.at[wid], idx_v)

        def body(j, carry):
            pltpu.async_copy(x_hbm.at[idx_v.at[j]], rows_v, sem).wait()
            pltpu.sync_copy(
                rows_v,
                out_hbm.at[pl.ds(wid * n_chunks * PCHUNK + j * PCHUNK, PCHUNK)])
            return carry

        lax.fori_loop(0, n_chunks, body, 0)

    return pair_gather


def _node_body(nf_ref, p0_ref, p1_ref, d0_ref, d1_ref, t0_ref, t1_ref,
               w1t_ref, b1_ref, out_ref):
    tail = t0_ref[COLSUM_BASE - N_NODES:, :] \
        + t1_ref[COLSUM_BASE - N_NODES:, :]
    mu = jnp.sum(tail, axis=0, keepdims=True) * (1.0 / N_NODES)
    s = p0_ref[0] + p1_ref[0]
    deg = d0_ref[0, :, 0:1] + d1_ref[0, :, 0:1]
    f = nf_ref[...]
    t = (f + mu) * (s + deg * mu) / jnp.maximum(deg, 1.0)
    out_ref[...] = jnp.dot(t, w1t_ref[...],
                           preferred_element_type=jnp.float32) + b1_ref[0:1, :]


def _head_body(p1_ref, p2_ref, wasp_ref, g_ref, gt_ref, wat_ref, wmg_ref,
               wc1_ref, wc2_ref, bc_ref, out_ref):
    def merge(p):
        q = jnp.dot(p, wasp_ref[...], preferred_element_type=jnp.float32)
        l = jnp.dot(q * wat_ref[0:1, :], g_ref[...],
                    preferred_element_type=jnp.float32)
        m = l - jnp.max(l, axis=1, keepdims=True)
        e = jnp.exp(m)
        sm = e / jnp.sum(e, axis=1, keepdims=True)
        ab = jnp.dot(sm, gt_ref[...], preferred_element_type=jnp.float32)
        return jnp.dot(ab * q * wmg_ref[0:1, :], g_ref[...],
                       preferred_element_type=jnp.float32)

    m1 = merge(p1_ref[...])
    m2 = merge(p2_ref[...])
    o = jnp.dot(m1, wc1_ref[...], preferred_element_type=jnp.float32) \
        + jnp.dot(m2, wc2_ref[...], preferred_element_type=jnp.float32) \
        + bc_ref[0:1, :]
    mx = jnp.max(o, axis=1, keepdims=True)
    out_ref[...] = o - mx - jnp.log(jnp.sum(jnp.exp(o - mx), axis=1,
                                            keepdims=True))


def kernel(input, node_feature, edge_index, W1, b1, Wasp, Wattn, battn,
           Wmerge, bmerge, Wc, bc):
    f32 = jnp.float32
    n_edges = edge_index.shape[1]

    fake_src = jnp.arange(N_NODES, dtype=jnp.int32)
    fake_dst = COLSUM_BASE + (fake_src % NCOLSUM)
    total = n_edges + N_NODES
    n_chunks = _edge_chunks(total)
    e_pad = NW * n_chunks * CHUNK
    npad = e_pad - total
    pad_src = jnp.zeros((npad,), jnp.int32)
    pad_dst = DUMMY + (jnp.arange(npad, dtype=jnp.int32) % NDUMMY)
    real = ((edge_index[1] << IDX_BITS) | edge_index[0]).reshape(NW, -1)
    tail = jnp.concatenate(
        [(fake_dst << IDX_BITS) | fake_src,
         (pad_dst << IDX_BITS) | pad_src]).reshape(NW, -1)
    packed = jnp.concatenate([real, tail], axis=1) \
        .reshape(NW, n_chunks, CHUNK)

    zero_f = jnp.zeros((ROWS_PER_TILE, IN_DIM), f32)
    zero_d = jnp.zeros((ROWS_PER_TILE, DEG_W), f32)
    ones_rows = jnp.zeros((CHUNK, DEG_W), f32).at[:, 0].set(1.0)
    pfeat, pdeg = _make_edge_scatter(n_chunks)(
        node_feature, packed, zero_f, zero_d, ones_rows)

    t0 = pfeat[0, N_NODES:]
    t1 = pfeat[1, N_NODES:]
    w1t = W1.T
    b1b = jnp.broadcast_to(b1[None, :], (8, HID_DIM))
    blk = 2000
    grid = N_NODES // blk
    x = pl.pallas_call(
        _node_body,
        grid=(grid,),
        in_specs=[
            pl.BlockSpec((blk, IN_DIM), lambda i: (i, 0)),
            pl.BlockSpec((1, blk, IN_DIM), lambda i: (0, i, 0)),
            pl.BlockSpec((1, blk, IN_DIM), lambda i: (1, i, 0)),
            pl.BlockSpec((1, blk, DEG_W), lambda i: (0, i, 0)),
            pl.BlockSpec((1, blk, DEG_W), lambda i: (1, i, 0)),
            pl.BlockSpec((N_PAD - N_NODES, IN_DIM), lambda i: (0, 0)),
            pl.BlockSpec((N_PAD - N_NODES, IN_DIM), lambda i: (0, 0)),
            pl.BlockSpec((IN_DIM, HID_DIM), lambda i: (0, 0)),
            pl.BlockSpec((8, HID_DIM), lambda i: (0, 0)),
        ],
        out_specs=pl.BlockSpec((blk, HID_DIM), lambda i: (i, 0)),
        out_shape=jax.ShapeDtypeStruct((N_NODES, HID_DIM), f32),
    )(node_feature, pfeat, pfeat, pdeg, pdeg, t0, t1, w1t, b1b)

    idx_flat = jnp.concatenate([input[:, 0], input[:, 1]]).astype(jnp.int32)
    idx_r = idx_flat.reshape(NW, (2 * BATCH) // (NW * PCHUNK), PCHUNK)
    pairs = _make_pair_gather()(x, idx_r)

    hid = NUM_ASPECT * ASPECT_DIM
    g = (jnp.arange(hid)[:, None] // ASPECT_DIM
         == jnp.arange(NUM_ASPECT)[None, :]).astype(f32)
    gt = g.T
    wat = jnp.broadcast_to(jnp.tile(Wattn[0], NUM_ASPECT)[None, :], (8, hid))
    wmg = jnp.broadcast_to(jnp.tile(Wmerge[0], NUM_ASPECT)[None, :], (8, hid))
    wc1 = Wc[:, :NUM_ASPECT].T
    wc2 = Wc[:, NUM_ASPECT:].T
    bc_eff = bc + bmerge[0] * jnp.sum(Wc, axis=1)
    bcb = jnp.broadcast_to(bc_eff[None, :], (8, 2))
    p_first = pairs[:BATCH]
    p_second = pairs[BATCH:]
    bblk = 1024
    bgrid = BATCH // bblk
    out = pl.pallas_call(
        _head_body,
        grid=(bgrid,),
        in_specs=[
            pl.BlockSpec((bblk, HID_DIM), lambda i: (i, 0)),
            pl.BlockSpec((bblk, HID_DIM), lambda i: (i, 0)),
            pl.BlockSpec((HID_DIM, hid), lambda i: (0, 0)),
            pl.BlockSpec((hid, NUM_ASPECT), lambda i: (0, 0)),
            pl.BlockSpec((NUM_ASPECT, hid), lambda i: (0, 0)),
            pl.BlockSpec((8, hid), lambda i: (0, 0)),
            pl.BlockSpec((8, hid), lambda i: (0, 0)),
            pl.BlockSpec((NUM_ASPECT, 2), lambda i: (0, 0)),
            pl.BlockSpec((NUM_ASPECT, 2), lambda i: (0, 0)),
            pl.BlockSpec((8, 2), lambda i: (0, 0)),
        ],
        out_specs=pl.BlockSpec((bblk, 2), lambda i: (i, 0)),
        out_shape=jax.ShapeDtypeStruct((BATCH, 2), f32),
    )(p_first, p_second, Wasp.T, g, gt, wat, wmg, wc1, wc2, bcb)
    return out


# speedup vs baseline: 1.3507x; 1.0106x over previous
import functools

import jax
import jax.numpy as jnp
from jax import lax
from jax.experimental import pallas as pl
from jax.experimental.pallas import tpu as pltpu
from jax.experimental.pallas import tpu_sc as plsc

N_NODES = 10000
IN_DIM = 128
HID_DIM = 256
NUM_ASPECT = 8
ASPECT_DIM = 32
BATCH = 4096

DEG_W = 16
N_PAD = 10112
NCOLSUM = 16
COLSUM_BASE = N_PAD - NCOLSUM
DUMMY = N_NODES
NDUMMY = 64

NC, NS = 2, 16
NW = NC * NS
CHUNK = 96
PCHUNK = 128
ROWS_PER_TILE = N_PAD // NS


NBUF = 2
IDX_BITS = 14


def _edge_chunks(n_edges_total):
    per_w = -(-n_edges_total // (NW * CHUNK))
    return -(-per_w // NBUF) * NBUF


def _make_edge_scatter(n_chunks):
    mesh = plsc.VectorSubcoreMesh(core_axis_name="c", subcore_axis_name="s")

    @functools.partial(
        pl.kernel,
        mesh=mesh,
        compiler_params=pltpu.CompilerParams(use_tc_tiling_on_sc=False),
        out_type=(jax.ShapeDtypeStruct((NC, N_PAD, IN_DIM), jnp.float32),
                  jax.ShapeDtypeStruct((NC, N_PAD, DEG_W), jnp.float32)),
        scratch_types=[
            pltpu.VMEM((n_chunks, CHUNK), jnp.int32),
            pltpu.VMEM((NBUF, CHUNK), jnp.int32),
            pltpu.VMEM((1, CHUNK), jnp.int32),
            pltpu.VMEM((NBUF, CHUNK, IN_DIM), jnp.float32),
            pltpu.VMEM((CHUNK, DEG_W), jnp.float32),
            pltpu.VMEM_SHARED((N_PAD, IN_DIM), jnp.float32),
            pltpu.VMEM_SHARED((N_PAD, DEG_W), jnp.float32),
            pltpu.SemaphoreType.DMA,
        ],
    )
    def edge_scatter(feat_hbm, edges_hbm, zerof_hbm, zerod_hbm, ones_hbm,
                     outf_hbm, outd_hbm,
                     packed_v, usrc_v, udst_v, rows_v, ones_v,
                     accf_sh, accd_sh, gsem):
        cid = lax.axis_index("c")
        sid = lax.axis_index("s")
        wid = sid * NC + cid
        r0 = sid * ROWS_PER_TILE
        pltpu.sync_copy(zerof_hbm, accf_sh.at[pl.ds(r0, ROWS_PER_TILE)])
        pltpu.sync_copy(zerod_hbm, accd_sh.at[pl.ds(r0, ROWS_PER_TILE)])
        pltpu.sync_copy(edges_hbm.at[wid], packed_v)
        pltpu.sync_copy(ones_hbm, ones_v)
        plsc.subcore_barrier()

        def unpack_src(j, b):
            for k in range(CHUNK // 16):
                w = packed_v[j, pl.ds(k * 16, 16)]
                usrc_v[b, pl.ds(k * 16, 16)] = w & ((1 << IDX_BITS) - 1)

        def unpack_dst(j):
            for k in range(CHUNK // 16):
                w = packed_v[j, pl.ds(k * 16, 16)]
                udst_v[0, pl.ds(k * 16, 16)] = w >> IDX_BITS

        for b in range(NBUF):
            unpack_src(b, b)
            pltpu.async_copy(feat_hbm.at[usrc_v.at[b]], rows_v.at[b], gsem)

        def group(g, carry):
            for b in range(NBUF):
                j = g * NBUF + b
                pltpu.make_async_copy(
                    feat_hbm.at[usrc_v.at[b]], rows_v.at[b], gsem).wait()
                unpack_dst(j)
                pltpu.sync_copy(rows_v.at[b], accf_sh.at[udst_v.at[0]],
                                add=True)
                pltpu.sync_copy(ones_v, accd_sh.at[udst_v.at[0]], add=True)
                nj = j + NBUF

                @pl.when(nj < n_chunks)
                def _issue():
                    unpack_src(nj, b)
                    pltpu.async_copy(
                        feat_hbm.at[usrc_v.at[b]], rows_v.at[b], gsem)
            return carry

        lax.fori_loop(0, n_chunks // NBUF, group, 0)
        plsc.subcore_barrier()
        pltpu.sync_copy(accf_sh.at[pl.ds(r0, ROWS_PER_TILE)],
                        outf_hbm.at[cid, pl.ds(r0, ROWS_PER_TILE)])
        pltpu.sync_copy(accd_sh.at[pl.ds(r0, ROWS_PER_TILE)],
                        outd_hbm.at[cid, pl.ds(r0, ROWS_PER_TILE)])

    return edge_scatter


def _make_pair_gather():
    mesh = plsc.VectorSubcoreMesh(core_axis_name="c", subcore_axis_name="s")
    n_chunks = (2 * BATCH) // (NW * PCHUNK)

    @functools.partial(
        pl.kernel,
        mesh=mesh,
        out_type=jax.ShapeDtypeStruct((2 * BATCH, HID_DIM), jnp.float32),
        scratch_types=[
            pltpu.VMEM((n_chunks, PCHUNK), jnp.int32),
            pltpu.VMEM((PCHUNK, HID_DIM), jnp.float32),
            pltpu.SemaphoreType.DMA,
        ],
    )
    def pair_gather(x_hbm, idx_hbm, out_hbm, idx_v, rows_v, sem):
        cid = lax.axis_index("c")
        sid = lax.axis_index("s")
        wid = sid * NC + cid
        pltpu.sync_copy(idx_hbm.at[wid], idx_v)

        def body(j, carry):
            pltpu.async_copy(x_hbm.at[idx_v.at[j]], rows_v, sem).wait()
            pltpu.sync_copy(
                rows_v,
                out_hbm.at[pl.ds(wid * n_chunks * PCHUNK + j * PCHUNK, PCHUNK)])
            return carry

        lax.fori_loop(0, n_chunks, body, 0)

    return pair_gather


def _node_body(nf_ref, p0_ref, p1_ref, d0_ref, d1_ref, t0_ref, t1_ref,
               w1t_ref, b1_ref, out_ref):
    tail = t0_ref[COLSUM_BASE - N_NODES:, :] \
        + t1_ref[COLSUM_BASE - N_NODES:, :]
    mu = jnp.sum(tail, axis=0, keepdims=True) * (1.0 / N_NODES)
    s = p0_ref[0] + p1_ref[0]
    deg = d0_ref[0, :, 0:1] + d1_ref[0, :, 0:1]
    f = nf_ref[...]
    t = (f + mu) * (s + deg * mu) / jnp.maximum(deg, 1.0)
    out_ref[...] = jnp.dot(t, w1t_ref[...],
                           preferred_element_type=jnp.float32) + b1_ref[0:1, :]


def _head_body(p1_ref, p2_ref, wasp_ref, g_ref, gt_ref, wat_ref, wmg_ref,
               wc1_ref, wc2_ref, bc_ref, out_ref):
    def merge(p):
        q = jnp.dot(p, wasp_ref[...], preferred_element_type=jnp.float32)
        l = jnp.dot(q * wat_ref[0:1, :], g_ref[...],
                    preferred_element_type=jnp.float32)
        m = l - jnp.max(l, axis=1, keepdims=True)
        e = jnp.exp(m)
        sm = e / jnp.sum(e, axis=1, keepdims=True)
        ab = jnp.dot(sm, gt_ref[...], preferred_element_type=jnp.float32)
        return jnp.dot(ab * q * wmg_ref[0:1, :], g_ref[...],
                       preferred_element_type=jnp.float32)

    m1 = merge(p1_ref[...])
    m2 = merge(p2_ref[...])
    o = jnp.dot(m1, wc1_ref[...], preferred_element_type=jnp.float32) \
        + jnp.dot(m2, wc2_ref[...], preferred_element_type=jnp.float32) \
        + bc_ref[0:1, :]
    mx = jnp.max(o, axis=1, keepdims=True)
    out_ref[...] = o - mx - jnp.log(jnp.sum(jnp.exp(o - mx), axis=1,
                                            keepdims=True))


def kernel(input, node_feature, edge_index, W1, b1, Wasp, Wattn, battn,
           Wmerge, bmerge, Wc, bc):
    f32 = jnp.float32
    n_edges = edge_index.shape[1]

    fake_src = jnp.arange(N_NODES, dtype=jnp.int32)
    fake_dst = COLSUM_BASE + (fake_src % NCOLSUM)
    total = n_edges + N_NODES
    n_chunks = _edge_chunks(total)
    e_pad = NW * n_chunks * CHUNK
    npad = e_pad - total
    pad_src = jnp.zeros((npad,), jnp.int32)
    pad_dst = DUMMY + (jnp.arange(npad, dtype=jnp.int32) % NDUMMY)
    real = ((edge_index[1] << IDX_BITS) | edge_index[0]).reshape(NW, -1)
    tail = jnp.concatenate(
        [(fake_dst << IDX_BITS) | fake_src,
         (pad_dst << IDX_BITS) | pad_src]).reshape(NW, -1)
    packed = jnp.concatenate([real, tail], axis=1) \
        .reshape(NW, n_chunks, CHUNK)

    zero_f = jnp.zeros((ROWS_PER_TILE, IN_DIM), f32)
    zero_d = jnp.zeros((ROWS_PER_TILE, DEG_W), f32)
    ones_rows = jnp.zeros((CHUNK, DEG_W), f32).at[:, 0].set(1.0)
    pfeat, pdeg = _make_edge_scatter(n_chunks)(
        node_feature, packed, zero_f, zero_d, ones_rows)

    t0 = pfeat[0, N_NODES:]
    t1 = pfeat[1, N_NODES:]
    w1t = W1.T
    b1b = jnp.broadcast_to(b1[None, :], (8, HID_DIM))
    blk = 5000
    grid = N_NODES // blk
    x = pl.pallas_call(
        _node_body,
        grid=(grid,),
        in_specs=[
            pl.BlockSpec((blk, IN_DIM), lambda i: (i, 0)),
            pl.BlockSpec((1, blk, IN_DIM), lambda i: (0, i, 0)),
            pl.BlockSpec((1, blk, IN_DIM), lambda i: (1, i, 0)),
            pl.BlockSpec((1, blk, DEG_W), lambda i: (0, i, 0)),
            pl.BlockSpec((1, blk, DEG_W), lambda i: (1, i, 0)),
            pl.BlockSpec((N_PAD - N_NODES, IN_DIM), lambda i: (0, 0)),
            pl.BlockSpec((N_PAD - N_NODES, IN_DIM), lambda i: (0, 0)),
            pl.BlockSpec((IN_DIM, HID_DIM), lambda i: (0, 0)),
            pl.BlockSpec((8, HID_DIM), lambda i: (0, 0)),
        ],
        out_specs=pl.BlockSpec((blk, HID_DIM), lambda i: (i, 0)),
        out_shape=jax.ShapeDtypeStruct((N_NODES, HID_DIM), f32),
    )(node_feature, pfeat, pfeat, pdeg, pdeg, t0, t1, w1t, b1b)

    idx_flat = jnp.concatenate([input[:, 0], input[:, 1]]).astype(jnp.int32)
    idx_r = idx_flat.reshape(NW, (2 * BATCH) // (NW * PCHUNK), PCHUNK)
    pairs = _make_pair_gather()(x, idx_r)

    hid = NUM_ASPECT * ASPECT_DIM
    g = (jnp.arange(hid)[:, None] // ASPECT_DIM
         == jnp.arange(NUM_ASPECT)[None, :]).astype(f32)
    gt = g.T
    wat = jnp.broadcast_to(jnp.tile(Wattn[0], NUM_ASPECT)[None, :], (8, hid))
    wmg = jnp.broadcast_to(jnp.tile(Wmerge[0], NUM_ASPECT)[None, :], (8, hid))
    wc1 = Wc[:, :NUM_ASPECT].T
    wc2 = Wc[:, NUM_ASPECT:].T
    bc_eff = bc + bmerge[0] * jnp.sum(Wc, axis=1)
    bcb = jnp.broadcast_to(bc_eff[None, :], (8, 2))
    p_first = pairs[:BATCH]
    p_second = pairs[BATCH:]
    bblk = 2048
    bgrid = BATCH // bblk
    out = pl.pallas_call(
        _head_body,
        grid=(bgrid,),
        in_specs=[
            pl.BlockSpec((bblk, HID_DIM), lambda i: (i, 0)),
            pl.BlockSpec((bblk, HID_DIM), lambda i: (i, 0)),
            pl.BlockSpec((HID_DIM, hid), lambda i: (0, 0)),
            pl.BlockSpec((hid, NUM_ASPECT), lambda i: (0, 0)),
            pl.BlockSpec((NUM_ASPECT, hid), lambda i: (0, 0)),
            pl.BlockSpec((8, hid), lambda i: (0, 0)),
            pl.BlockSpec((8, hid), lambda i: (0, 0)),
            pl.BlockSpec((NUM_ASPECT, 2), lambda i: (0, 0)),
            pl.BlockSpec((NUM_ASPECT, 2), lambda i: (0, 0)),
            pl.BlockSpec((8, 2), lambda i: (0, 0)),
        ],
        out_specs=pl.BlockSpec((bblk, 2), lambda i: (i, 0)),
        out_shape=jax.ShapeDtypeStruct((BATCH, 2), f32),
    )(p_first, p_second, Wasp.T, g, gt, wat, wmg, wc1, wc2, bcb)
    return out
